# baseline (speedup 1.0000x reference)
import functools

import jax
import jax.numpy as jnp
from jax import lax
from jax.experimental import pallas as pl
from jax.experimental.pallas import tpu as pltpu
from jax.experimental.pallas import tpu_sc as plsc

VOCAB = 1000000
EMBED_DIM = 64
BATCH = 4096
SEQ = 200
B_TOTAL = BATCH * SEQ

NUM_CORES = 2
NUM_SUBCORES = 16
NUM_WORKERS = NUM_CORES * NUM_SUBCORES
ROWS_PER_W = BATCH // NUM_WORKERS

NBUF = 4
N_GROUPS = ROWS_PER_W // NBUF
assert ROWS_PER_W % NBUF == 0


def _make_emb_kernel():
    mesh = plsc.VectorSubcoreMesh(core_axis_name="c", subcore_axis_name="s")

    scratch = (
        [pltpu.VMEM((SEQ,), jnp.int32) for _ in range(NBUF)]
        + [pltpu.VMEM((SEQ, EMBED_DIM), jnp.float32) for _ in range(NBUF)]
        + [pltpu.SemaphoreType.DMA for _ in range(3 * NBUF)]
    )

    @functools.partial(
        pl.kernel,
        mesh=mesh,
        out_type=jax.ShapeDtypeStruct((BATCH, SEQ, EMBED_DIM), jnp.float32),
        compiler_params=pltpu.CompilerParams(use_tc_tiling_on_sc=False),
        scratch_types=scratch,
    )
    def emb_kernel(idx_hbm, table_hbm, out_hbm, *scr):
        idx_vs = scr[:NBUF]
        rows_vs = scr[NBUF : 2 * NBUF]
        idx_sems = scr[2 * NBUF : 3 * NBUF]
        gat_sems = scr[3 * NBUF : 4 * NBUF]
        out_sems = scr[4 * NBUF : 5 * NBUF]

        wid = lax.axis_index("s") * NUM_CORES + lax.axis_index("c")
        row0 = wid * ROWS_PER_W

        for b in range(NBUF):
            pltpu.async_copy(
                idx_hbm.at[pl.ds((row0 + b) * SEQ, SEQ)], idx_vs[b], idx_sems[b]
            )

        def group_body(g, carry):
            row_g = row0 + g * NBUF
            for b in range(NBUF):
                pltpu.make_async_copy(
                    idx_hbm.at[pl.ds((row_g + b) * SEQ, SEQ)],
                    idx_vs[b],
                    idx_sems[b],
                ).wait()
                pltpu.async_copy(
                    table_hbm.at[idx_vs[b]], rows_vs[b], gat_sems[b]
                )
            for b in range(NBUF):
                pltpu.make_async_copy(
                    table_hbm.at[idx_vs[b]], rows_vs[b], gat_sems[b]
                ).wait()
                pltpu.async_copy(
                    rows_vs[b], out_hbm.at[row_g + b], out_sems[b]
                )

                @pl.when(g + 1 < N_GROUPS)
                def _prefetch(b=b, row_g=row_g):
                    pltpu.async_copy(
                        idx_hbm.at[pl.ds((row_g + NBUF + b) * SEQ, SEQ)],
                        idx_vs[b],
                        idx_sems[b],
                    )

            for b in range(NBUF):
                pltpu.make_async_copy(
                    rows_vs[b], out_hbm.at[row_g + b], out_sems[b]
                ).wait()
            return carry

        lax.fori_loop(0, N_GROUPS, group_body, 0)

    return emb_kernel


_emb = _make_emb_kernel()


def kernel(x, table):
    idx = x.reshape(-1).astype(jnp.int32)
    return _emb(idx, table)


# speedup vs baseline: 1.0319x; 1.0319x over previous
import functools

import jax
import jax.numpy as jnp
from jax import lax
from jax.experimental import pallas as pl
from jax.experimental.pallas import tpu as pltpu
from jax.experimental.pallas import tpu_sc as plsc

VOCAB = 1000000
EMBED_DIM = 64
EMBED_PAD = 128
BATCH = 4096
SEQ = 200
B_TOTAL = BATCH * SEQ

NUM_CORES = 2
NUM_SUBCORES = 16
NUM_WORKERS = NUM_CORES * NUM_SUBCORES
B_PER_W = B_TOTAL // NUM_WORKERS

NBUF = 2
CHUNK = 400
GROUP = NBUF * CHUNK
N_GROUPS = B_PER_W // GROUP
assert B_PER_W % GROUP == 0

TBLK = 1024


def _tp_body(tin_ref, tout_ref):
    t = jnp.transpose(tin_ref[...], (1, 0))
    tout_ref[...] = jnp.concatenate([t, jnp.zeros_like(t)], axis=1)


_transpose_pad = pl.pallas_call(
    _tp_body,
    grid=(pl.cdiv(VOCAB, TBLK),),
    in_specs=[pl.BlockSpec((EMBED_DIM, TBLK), lambda i: (0, i))],
    out_specs=pl.BlockSpec((TBLK, EMBED_PAD), lambda i: (i, 0)),
    out_shape=jax.ShapeDtypeStruct((VOCAB, EMBED_PAD), jnp.float32),
)


def _make_emb_kernel():
    mesh = plsc.VectorSubcoreMesh(core_axis_name="c", subcore_axis_name="s")

    scratch = (
        [pltpu.VMEM((CHUNK,), jnp.int32) for _ in range(NBUF)]
        + [pltpu.VMEM((CHUNK, EMBED_PAD), jnp.float32) for _ in range(NBUF)]
        + [pltpu.SemaphoreType.DMA for _ in range(3 * NBUF)]
    )

    @functools.partial(
        pl.kernel,
        mesh=mesh,
        out_type=jax.ShapeDtypeStruct((B_TOTAL, EMBED_PAD), jnp.float32),
        scratch_types=scratch,
    )
    def emb_kernel(idx_hbm, table_hbm, out_hbm, *scr):
        idx_vs = scr[:NBUF]
        rows_vs = scr[NBUF : 2 * NBUF]
        idx_sems = scr[2 * NBUF : 3 * NBUF]
        gat_sems = scr[3 * NBUF : 4 * NBUF]
        out_sems = scr[4 * NBUF : 5 * NBUF]

        wid = lax.axis_index("s") * NUM_CORES + lax.axis_index("c")
        base0 = wid * B_PER_W

        for b in range(NBUF):
            pltpu.async_copy(
                idx_hbm.at[pl.ds(base0 + b * CHUNK, CHUNK)], idx_vs[b], idx_sems[b]
            )

        def group_body(g, carry):
            base_g = base0 + g * GROUP
            for b in range(NBUF):
                pltpu.make_async_copy(
                    idx_hbm.at[pl.ds(base_g + b * CHUNK, CHUNK)],
                    idx_vs[b],
                    idx_sems[b],
                ).wait()
                pltpu.async_copy(
                    table_hbm.at[idx_vs[b]], rows_vs[b], gat_sems[b]
                )
            for b in range(NBUF):
                chunk_base = base_g + b * CHUNK
                pltpu.make_async_copy(
                    table_hbm.at[idx_vs[b]], rows_vs[b], gat_sems[b]
                ).wait()
                pltpu.async_copy(
                    rows_vs[b],
                    out_hbm.at[pl.ds(chunk_base, CHUNK)],
                    out_sems[b],
                )

                @pl.when(g + 1 < N_GROUPS)
                def _prefetch(b=b, base_g=base_g):
                    pltpu.async_copy(
                        idx_hbm.at[pl.ds(base_g + GROUP + b * CHUNK, CHUNK)],
                        idx_vs[b],
                        idx_sems[b],
                    )

            for b in range(NBUF):
                pltpu.make_async_copy(
                    rows_vs[b],
                    out_hbm.at[pl.ds(base_g + b * CHUNK, CHUNK)],
                    out_sems[b],
                ).wait()
            return carry

        lax.fori_loop(0, N_GROUPS, group_body, 0)

    return emb_kernel


_emb = _make_emb_kernel()


def kernel(x, table):
    idx = x.reshape(-1).astype(jnp.int32)
    table_pad = _transpose_pad(table.T)
    out_pad = _emb(idx, table_pad)
    return out_pad[:, :EMBED_DIM].reshape(BATCH, SEQ, EMBED_DIM)


# speedup vs baseline: 1.2263x; 1.1884x over previous
import functools

import jax
import jax.numpy as jnp
from jax import lax
from jax.experimental import pallas as pl
from jax.experimental.pallas import tpu as pltpu
from jax.experimental.pallas import tpu_sc as plsc

VOCAB = 1000000
EMBED_DIM = 64
EMBED_PAD = 128
BATCH = 4096
SEQ = 200
B_TOTAL = BATCH * SEQ

NUM_CORES = 2
NUM_SUBCORES = 16
NUM_WORKERS = NUM_CORES * NUM_SUBCORES
B_PER_W = B_TOTAL // NUM_WORKERS

NBUF = 2
CHUNK = 400
GROUP = NBUF * CHUNK
N_GROUPS = B_PER_W // GROUP
assert B_PER_W % GROUP == 0


def _make_emb_kernel():
    mesh = plsc.VectorSubcoreMesh(core_axis_name="c", subcore_axis_name="s")

    scratch = (
        [pltpu.VMEM((CHUNK,), jnp.int32) for _ in range(NBUF)]
        + [pltpu.VMEM((CHUNK, EMBED_PAD), jnp.float32) for _ in range(NBUF)]
        + [pltpu.SemaphoreType.DMA for _ in range(3 * NBUF)]
    )

    @functools.partial(
        pl.kernel,
        mesh=mesh,
        out_type=jax.ShapeDtypeStruct((B_TOTAL, EMBED_PAD), jnp.float32),
        scratch_types=scratch,
    )
    def emb_kernel(idx_hbm, table_hbm, out_hbm, *scr):
        idx_vs = scr[:NBUF]
        rows_vs = scr[NBUF : 2 * NBUF]
        idx_sems = scr[2 * NBUF : 3 * NBUF]
        gat_sems = scr[3 * NBUF : 4 * NBUF]
        out_sems = scr[4 * NBUF : 5 * NBUF]

        wid = lax.axis_index("s") * NUM_CORES + lax.axis_index("c")
        base0 = wid * B_PER_W

        for b in range(NBUF):
            pltpu.async_copy(
                idx_hbm.at[pl.ds(base0 + b * CHUNK, CHUNK)], idx_vs[b], idx_sems[b]
            )

        def group_body(g, carry):
            base_g = base0 + g * GROUP
            for b in range(NBUF):
                pltpu.make_async_copy(
                    idx_hbm.at[pl.ds(base_g + b * CHUNK, CHUNK)],
                    idx_vs[b],
                    idx_sems[b],
                ).wait()
                pltpu.async_copy(
                    table_hbm.at[idx_vs[b]], rows_vs[b], gat_sems[b]
                )
            for b in range(NBUF):
                chunk_base = base_g + b * CHUNK
                pltpu.make_async_copy(
                    table_hbm.at[idx_vs[b]], rows_vs[b], gat_sems[b]
                ).wait()
                pltpu.async_copy(
                    rows_vs[b],
                    out_hbm.at[pl.ds(chunk_base, CHUNK)],
                    out_sems[b],
                )

                @pl.when(g + 1 < N_GROUPS)
                def _prefetch(b=b, base_g=base_g):
                    pltpu.async_copy(
                        idx_hbm.at[pl.ds(base_g + GROUP + b * CHUNK, CHUNK)],
                        idx_vs[b],
                        idx_sems[b],
                    )

            for b in range(NBUF):
                pltpu.make_async_copy(
                    rows_vs[b],
                    out_hbm.at[pl.ds(base_g + b * CHUNK, CHUNK)],
                    out_sems[b],
                ).wait()
            return carry

        lax.fori_loop(0, N_GROUPS, group_body, 0)

    return emb_kernel


_emb = _make_emb_kernel()


def kernel(x, table):
    idx = x.reshape(-1).astype(jnp.int32)
    table_pad = jnp.pad(table, ((0, 0), (0, EMBED_PAD - EMBED_DIM)))
    out_pad = _emb(idx, table_pad)
    return out_pad[:, :EMBED_DIM].reshape(BATCH, SEQ, EMBED_DIM)
